# baseline (speedup 1.0000x reference)
import functools

import jax
import jax.numpy as jnp
from jax import lax
from jax.experimental import pallas as pl
from jax.experimental.pallas import tpu as pltpu
from jax.experimental.pallas import tpu_sc as plsc

EMBED_DIM = 64
ROWS, COLS = 16384, 200
NC, NS = 2, 16
NW = NC * NS
R_PER_W = ROWS // NW
NBUF = 3
HEAD = 5
TAIL = 3
N_GRP = (R_PER_W - HEAD - TAIL) // NBUF

_mesh = plsc.VectorSubcoreMesh(core_axis_name="c", subcore_axis_name="s")


@functools.partial(
    pl.kernel,
    out_type=jax.ShapeDtypeStruct((ROWS, COLS, EMBED_DIM), jnp.float32),
    mesh=_mesh,
    scratch_types=[
        pltpu.VMEM((COLS,), jnp.int32),
        pltpu.VMEM((COLS,), jnp.int32),
        pltpu.VMEM((COLS,), jnp.int32),
        pltpu.VMEM((COLS, EMBED_DIM), jnp.float32),
        pltpu.VMEM((COLS, EMBED_DIM), jnp.float32),
        pltpu.VMEM((COLS, EMBED_DIM), jnp.float32),
        pltpu.SemaphoreType.DMA,
        pltpu.SemaphoreType.DMA,
        pltpu.SemaphoreType.DMA,
        pltpu.SemaphoreType.DMA,
        pltpu.SemaphoreType.DMA,
        pltpu.SemaphoreType.DMA,
        pltpu.SemaphoreType.DMA,
        pltpu.SemaphoreType.DMA,
        pltpu.SemaphoreType.DMA,
    ],
    compiler_params=pltpu.CompilerParams(use_tc_tiling_on_sc=False),
)
def _gather(tok_hbm, table_hbm, out_hbm,
            idx0, idx1, idx2, rows0, rows1, rows2,
            si0, si1, si2, sg0, sg1, sg2, sw0, sw1, sw2):
    idx = (idx0, idx1, idx2)
    rows = (rows0, rows1, rows2)
    si = (si0, si1, si2)
    sg = (sg0, sg1, sg2)
    sw = (sw0, sw1, sw2)

    wid = lax.axis_index("s") * NC + lax.axis_index("c")
    base = wid * R_PER_W

    def issue_idx(b, i):
        pltpu.async_copy(tok_hbm.at[base + i], idx[b], si[b])

    def wait_idx(b):
        pltpu.make_async_copy(tok_hbm.at[0], idx[b], si[b]).wait()

    def fire_gather(b):
        pltpu.async_copy(table_hbm.at[idx[b]], rows[b], sg[b])

    def wait_gather(b):
        pltpu.make_async_copy(table_hbm.at[idx[b]], rows[b], sg[b]).wait()

    def issue_wb(b, i):
        pltpu.async_copy(rows[b], out_hbm.at[base + i], sw[b])

    def wait_wb(b):
        pltpu.make_async_copy(rows[b], out_hbm.at[0], sw[b]).wait()

    def step(i, b, bj, need_wb_wait, has_prev, do_prefetch):
        wait_idx(b)
        if need_wb_wait:
            wait_wb(b)
        fire_gather(b)
        if has_prev:
            wait_gather(bj)
            issue_wb(bj, i - 1)
            if do_prefetch:
                issue_idx(bj, i + NBUF - 1)

    for b in range(NBUF):
        issue_idx(b, b)

    for i in range(HEAD):
        step(i, i % NBUF, (i - 1) % NBUF, i >= NBUF, i >= 1, True)

    def body(g, carry):
        for u in range(NBUF):
            i = HEAD + g * NBUF + u
            step(i, (HEAD + u) % NBUF, (HEAD + u - 1) % NBUF, True, True, True)
        return carry

    lax.fori_loop(0, N_GRP, body, 0)

    for i in range(R_PER_W - TAIL, R_PER_W):
        step(i, i % NBUF, (i - 1) % NBUF, True, True, i + NBUF - 1 < R_PER_W)

    last = R_PER_W - 1
    wait_gather(last % NBUF)
    issue_wb(last % NBUF, last)
    for i in range(R_PER_W - NBUF, R_PER_W):
        wait_wb(i % NBUF)


def kernel(tokens, table):
    return _gather(tokens.astype(jnp.int32), table)


# speedup vs baseline: 1.0033x; 1.0033x over previous
import functools

import jax
import jax.numpy as jnp
from jax import lax
from jax.experimental import pallas as pl
from jax.experimental.pallas import tpu as pltpu
from jax.experimental.pallas import tpu_sc as plsc

EMBED_DIM = 64
ROWS, COLS = 16384, 200
NC, NS = 2, 16
NW = NC * NS
R_PER_W = ROWS // NW
NBUF = 6
LAG = NBUF - 2
HEAD = NBUF
TAIL = 2
N_GRP = (R_PER_W - HEAD - TAIL) // NBUF

_mesh = plsc.VectorSubcoreMesh(core_axis_name="c", subcore_axis_name="s")


@functools.partial(
    pl.kernel,
    out_type=jax.ShapeDtypeStruct((ROWS, COLS, EMBED_DIM), jnp.float32),
    mesh=_mesh,
    scratch_types=(
        [pltpu.VMEM((COLS,), jnp.int32) for _ in range(NBUF)]
        + [pltpu.VMEM((COLS, EMBED_DIM), jnp.float32) for _ in range(NBUF)]
        + [pltpu.SemaphoreType.DMA for _ in range(3 * NBUF)]
    ),
    compiler_params=pltpu.CompilerParams(use_tc_tiling_on_sc=False),
)
def _gather(tok_hbm, table_hbm, out_hbm, *scratch):
    idx = scratch[:NBUF]
    rows = scratch[NBUF:2 * NBUF]
    si = scratch[2 * NBUF:3 * NBUF]
    sg = scratch[3 * NBUF:4 * NBUF]
    sw = scratch[4 * NBUF:5 * NBUF]

    wid = lax.axis_index("s") * NC + lax.axis_index("c")
    base = wid * R_PER_W

    def issue_idx(b, i):
        pltpu.async_copy(tok_hbm.at[base + i], idx[b], si[b])

    def wait_idx(b):
        pltpu.make_async_copy(tok_hbm.at[0], idx[b], si[b]).wait()

    def fire_gather(b):
        pltpu.async_copy(table_hbm.at[idx[b]], rows[b], sg[b])

    def wait_gather(b):
        pltpu.make_async_copy(table_hbm.at[idx[b]], rows[b], sg[b]).wait()

    def issue_wb(b, i):
        pltpu.async_copy(rows[b], out_hbm.at[base + i], sw[b])

    def wait_wb(b):
        pltpu.make_async_copy(rows[b], out_hbm.at[0], sw[b]).wait()

    def do_step(i, b, bj, need_wb_wait, has_drain, do_prefetch):
        wait_idx(b)
        if need_wb_wait:
            wait_wb(b)
        fire_gather(b)
        if has_drain:
            j = i - LAG
            wait_gather(bj)
            issue_wb(bj, j)
            if do_prefetch:
                issue_idx(bj, j + NBUF)

    for b in range(NBUF):
        issue_idx(b, b)

    for i in range(HEAD):
        do_step(i, i % NBUF, (i - LAG) % NBUF, i >= NBUF, i >= LAG, True)

    def body(g, carry):
        for u in range(NBUF):
            i = HEAD + g * NBUF + u
            do_step(i, (HEAD + u) % NBUF, (HEAD + u - LAG) % NBUF,
                    True, True, True)
        return carry

    lax.fori_loop(0, N_GRP, body, 0)

    for i in range(R_PER_W - TAIL, R_PER_W):
        do_step(i, i % NBUF, (i - LAG) % NBUF, True, True,
                i - LAG + NBUF < R_PER_W)

    for j in range(R_PER_W - LAG, R_PER_W):
        wait_gather(j % NBUF)
        issue_wb(j % NBUF, j)
    for j in range(R_PER_W - NBUF, R_PER_W):
        wait_wb(j % NBUF)


def kernel(tokens, table):
    return _gather(tokens.astype(jnp.int32), table)


# speedup vs baseline: 1.3030x; 1.2987x over previous
import functools

import jax
import jax.numpy as jnp
from jax import lax
from jax.experimental import pallas as pl
from jax.experimental.pallas import tpu as pltpu
from jax.experimental.pallas import tpu_sc as plsc

EMBED_DIM = 64
PAD_DIM = 128
ROWS, COLS = 16384, 200
NC, NS = 2, 16
NW = NC * NS
R_PER_W = ROWS // NW
NBUF = 4
LAG = NBUF - 2
HEAD = NBUF
TAIL = 4
N_GRP = (R_PER_W - HEAD - TAIL) // NBUF

_mesh = plsc.VectorSubcoreMesh(core_axis_name="c", subcore_axis_name="s")


@functools.partial(
    pl.kernel,
    out_type=jax.ShapeDtypeStruct((ROWS, COLS, PAD_DIM), jnp.float32),
    mesh=_mesh,
    scratch_types=(
        [pltpu.VMEM((COLS,), jnp.int32) for _ in range(NBUF)]
        + [pltpu.VMEM((COLS, PAD_DIM), jnp.float32) for _ in range(NBUF)]
        + [pltpu.SemaphoreType.DMA for _ in range(3 * NBUF)]
    ),
    compiler_params=pltpu.CompilerParams(use_tc_tiling_on_sc=True),
)
def _gather(tok_hbm, table_hbm, out_hbm, *scratch):
    idx = scratch[:NBUF]
    rows = scratch[NBUF:2 * NBUF]
    si = scratch[2 * NBUF:3 * NBUF]
    sg = scratch[3 * NBUF:4 * NBUF]
    sw = scratch[4 * NBUF:5 * NBUF]

    wid = lax.axis_index("s") * NC + lax.axis_index("c")
    base = wid * R_PER_W

    def issue_idx(b, i):
        pltpu.async_copy(tok_hbm.at[base + i], idx[b], si[b])

    def wait_idx(b):
        pltpu.make_async_copy(tok_hbm.at[0], idx[b], si[b]).wait()

    def fire_gather(b):
        pltpu.async_copy(table_hbm.at[idx[b]], rows[b], sg[b])

    def wait_gather(b):
        pltpu.make_async_copy(table_hbm.at[idx[b]], rows[b], sg[b]).wait()

    def issue_wb(b, i):
        pltpu.async_copy(rows[b], out_hbm.at[base + i], sw[b])

    def wait_wb(b):
        pltpu.make_async_copy(rows[b], out_hbm.at[0], sw[b]).wait()

    def do_step(i, b, bj, need_wb_wait, has_drain, do_prefetch):
        wait_idx(b)
        if need_wb_wait:
            wait_wb(b)
        fire_gather(b)
        if has_drain:
            j = i - LAG
            wait_gather(bj)
            issue_wb(bj, j)
            if do_prefetch:
                issue_idx(bj, j + NBUF)

    for b in range(NBUF):
        issue_idx(b, b)

    for i in range(HEAD):
        do_step(i, i % NBUF, (i - LAG) % NBUF, i >= NBUF, i >= LAG, True)

    def body(g, carry):
        for u in range(NBUF):
            i = HEAD + g * NBUF + u
            do_step(i, (HEAD + u) % NBUF, (HEAD + u - LAG) % NBUF,
                    True, True, True)
        return carry

    lax.fori_loop(0, N_GRP, body, 0)

    for i in range(R_PER_W - TAIL, R_PER_W):
        do_step(i, i % NBUF, (i - LAG) % NBUF, True, True,
                i - LAG + NBUF < R_PER_W)

    for j in range(R_PER_W - LAG, R_PER_W):
        wait_gather(j % NBUF)
        issue_wb(j % NBUF, j)
    for j in range(R_PER_W - NBUF, R_PER_W):
        wait_wb(j % NBUF)


def kernel(tokens, table):
    table_padded = jnp.pad(table, ((0, 0), (0, PAD_DIM - EMBED_DIM)))
    out = _gather(tokens.astype(jnp.int32), table_padded)
    return out[..., :EMBED_DIM]
